# baseline (speedup 1.0000x reference)
import functools

import jax
import jax.numpy as jnp
from jax import lax
from jax.experimental import pallas as pl
from jax.experimental.pallas import tpu as pltpu
from jax.experimental.pallas import tpu_sc as plsc

_B, _C, _H, _W = 4, 96, 384, 384
_NW = 32
_PLANES = _B * _C
_PPW = _PLANES // _NW
_CH_ROWS = 96
_CHUNKS = _PPW * 2
_NBUF = 3
_LANES = 16


def _sc_reduce_body(bt_hbm, out_hbm, idx0, idx1, idx2, buf0, buf1, buf2,
                    pacc_v, sem0, sem1, sem2):
    wid = lax.axis_index("s") * 2 + lax.axis_index("c")
    w12 = wid * _PPW
    liota = lax.iota(jnp.int32, _LANES)
    zeros = jnp.zeros((_LANES,), jnp.float32)

    slots = ((idx0, buf0, sem0), (idx1, buf1, sem1), (idx2, buf2, sem2))

    for j in range(_PPW):
        pacc_v[j] = zeros

    def fill_idx(idx_ref, k):
        plane = w12 + k // 2
        base = plane * _H + (k % 2) * (2 * _CH_ROWS)
        for j in range(_CH_ROWS // _LANES):
            idx_ref[pl.ds(j * _LANES, _LANES)] = (
                base + 2 * (j * _LANES) + 2 * liota)

    def start_gather(slot, k):
        idx_ref, buf_ref, sem = slot
        fill_idx(idx_ref, k)
        pltpu.make_async_copy(bt_hbm.at[idx_ref], buf_ref, sem).start()

    colvecs = [2 * liota + 32 * j for j in range(_W // (2 * _LANES))]

    def consume(slot, k):
        idx_ref, buf_ref, sem = slot
        pltpu.make_async_copy(bt_hbm.at[idx_ref], buf_ref, sem).wait()

        def rbody(r, accs):
            rowv = jnp.full((_LANES,), r, dtype=jnp.int32)
            accs = list(accs)
            for j, colv in enumerate(colvecs):
                v = plsc.load_gather(buf_ref, [rowv, colv])
                accs[j % 4] = accs[j % 4] + v
            return tuple(accs)

        a0, a1, a2, a3 = lax.fori_loop(0, _CH_ROWS, rbody,
                                       (zeros, zeros, zeros, zeros))
        pj = k // 2
        pacc_v[pj] = pacc_v[pj] + ((a0 + a1) + (a2 + a3))

    for b in range(_NBUF):
        start_gather(slots[b], jnp.int32(b))

    n_groups = _CHUNKS // _NBUF - 1

    def gbody(g, carry):
        for b in range(_NBUF):
            k = g * _NBUF + b
            consume(slots[b], k)
            start_gather(slots[b], k + _NBUF)
        return carry

    lax.fori_loop(0, n_groups, gbody, jnp.int32(0))

    for b in range(_NBUF):
        k = n_groups * _NBUF + b
        consume(slots[b], jnp.int32(k))

    pltpu.sync_copy(pacc_v, out_hbm.at[wid])


def _sc_pool_partials(b):
    bt = b.reshape(_PLANES * _H, _W)
    mesh = plsc.VectorSubcoreMesh(core_axis_name="c", subcore_axis_name="s")
    run = functools.partial(
        pl.kernel,
        out_type=jax.ShapeDtypeStruct((_NW, _PPW, _LANES), jnp.float32),
        mesh=mesh,
        scratch_types=[
            pltpu.VMEM((_CH_ROWS,), jnp.int32),
            pltpu.VMEM((_CH_ROWS,), jnp.int32),
            pltpu.VMEM((_CH_ROWS,), jnp.int32),
            pltpu.VMEM((_CH_ROWS, _W), jnp.float32),
            pltpu.VMEM((_CH_ROWS, _W), jnp.float32),
            pltpu.VMEM((_CH_ROWS, _W), jnp.float32),
            pltpu.VMEM((_PPW, _LANES), jnp.float32),
            pltpu.SemaphoreType.DMA,
            pltpu.SemaphoreType.DMA,
            pltpu.SemaphoreType.DMA,
        ],
        compiler_params=pltpu.CompilerParams(needs_layout_passes=False),
    )(_sc_reduce_body)
    return run(bt)


def _mlp_body(part_ref, fc1_ref, fc2_ref, o_ref):
    part = part_ref[...]
    pooled = jnp.sum(part, axis=2)
    pooled = pooled * (1.0 / (_H * _W))
    h = lax.dot_general(pooled, fc1_ref[...], (((1,), (1,)), ((), ())),
                        preferred_element_type=jnp.float32)
    h = jnp.maximum(h, 0.0)
    o_ref[...] = lax.dot_general(h, fc2_ref[...], (((1,), (1,)), ((), ())),
                                 preferred_element_type=jnp.float32)


def kernel(a, b, attn_w, attn_b, fc1_w, fc2_w):
    del a, attn_w, attn_b
    partials = _sc_pool_partials(b).reshape(_B, _C, _LANES)
    fc1 = fc1_w.reshape(_C // 4, _C)
    fc2 = fc2_w.reshape(_C, _C // 4)
    out = pl.pallas_call(
        _mlp_body,
        out_shape=jax.ShapeDtypeStruct((_B, _C), jnp.float32),
    )(partials, fc1, fc2)
    return out.reshape(_B, _C, 1, 1)
